# baseline (speedup 1.0000x reference)
import functools

import jax
import jax.numpy as jnp
from jax import lax
from jax.experimental import pallas as pl
from jax.experimental.pallas import tpu as pltpu
from jax.experimental.pallas import tpu_sc as plsc

N = 10000
E = 320000
A = 20000
D_IN = 128
D_H = 64
GW = 64
EMBW = 128
MAX_ARMY = 50

NC = 2
NS = 16
NW = NC * NS
NPAD = 10240
NSL = NPAD // NS
EPT = E // NW
BLK = 125
NB = EPT // BLK
DBLK = 80
DNB = EPT // DBLK
NBUF = 4
NG = NB // NBUF
APAD = 20480
ASL = APAD // NW
ABLK = 128
ANB = ASL // ABLK

_F32 = jnp.float32
_I32 = jnp.int32

_mesh = plsc.VectorSubcoreMesh(core_axis_name="c", subcore_axis_name="s")
_sc_params = pltpu.CompilerParams(use_tc_tiling_on_sc=False,
                                  needs_layout_passes=False)


@functools.partial(
    pl.kernel,
    out_type=jax.ShapeDtypeStruct((NC * NPAD,), _F32),
    mesh=_mesh,
    compiler_params=_sc_params,
    scratch_types=[
        pltpu.VMEM((DNB, DBLK), _I32),
        pltpu.VMEM((DBLK,), _F32),
        pltpu.VMEM((NSL,), _F32),
        pltpu.VMEM_SHARED((NPAD,), _F32),
    ],
)
def _sc_deg(dst_hbm, zeros1_hbm, out_hbm, idx_v, ones_v, obuf_v, acc_sh):
    c = lax.axis_index("c")
    s = lax.axis_index("s")
    w = s * NC + c
    pltpu.sync_copy(dst_hbm.at[w], idx_v)
    for k in range(DBLK // 16):
        ones_v[pl.ds(k * 16, 16)] = jnp.ones((16,), _F32)
    pltpu.sync_copy(zeros1_hbm, acc_sh.at[pl.ds(s * NSL, NSL)])
    plsc.subcore_barrier()

    def body(j, carry):
        pltpu.sync_copy(ones_v, acc_sh.at[idx_v.at[j]], add=True)
        return carry

    lax.fori_loop(0, DNB, body, 0)
    plsc.subcore_barrier()
    pltpu.sync_copy(acc_sh.at[pl.ds(s * NSL, NSL)], obuf_v)
    pltpu.sync_copy(obuf_v, out_hbm.at[pl.ds(c * NPAD + s * NSL, NSL)])


HGW = 32


@functools.partial(
    pl.kernel,
    out_type=jax.ShapeDtypeStruct((NPAD, 2 * GW), _F32),
    mesh=_mesh,
    compiler_params=_sc_params,
    scratch_types=[
        pltpu.VMEM((NB, BLK), _I32),
        pltpu.VMEM((NB, BLK), _I32),
        pltpu.VMEM((BLK, HGW), _F32),
        pltpu.VMEM((BLK, HGW), _F32),
        pltpu.VMEM((BLK, HGW), _F32),
        pltpu.VMEM((BLK, HGW), _F32),
        pltpu.VMEM_SHARED((NPAD, HGW), _F32),
        pltpu.VMEM_SHARED((NPAD, HGW), _F32),
        pltpu.SemaphoreType.DMA,
        pltpu.SemaphoreType.DMA,
        pltpu.SemaphoreType.DMA,
        pltpu.SemaphoreType.DMA,
        pltpu.SemaphoreType.DMA,
        pltpu.SemaphoreType.DMA,
        pltpu.SemaphoreType.DMA,
        pltpu.SemaphoreType.DMA,
    ],
)
def _sc_segsum(g_hbm, src_hbm, dst_hbm, zeros2_hbm, out_hbm,
               src_v, dst_v, b0, b1, b2, b3, tab_sh, acc_sh,
               gs0, gs1, gs2, gs3, ss0, ss1, ss2, ss3):
    c = lax.axis_index("c")
    s = lax.axis_index("s")
    w = s * NC + c
    bufs = (b0, b1, b2, b3)
    gsem = (gs0, gs1, gs2, gs3)
    ssem = (ss0, ss1, ss2, ss3)
    pltpu.sync_copy(src_hbm.at[w], src_v)
    pltpu.sync_copy(dst_hbm.at[w], dst_v)

    for p in range(2):
        pltpu.sync_copy(g_hbm.at[pl.ds(s * NSL, NSL), pl.ds(p * HGW, HGW)],
                        tab_sh.at[pl.ds(s * NSL, NSL)])
        for k in range(NSL // 128):
            pltpu.sync_copy(zeros2_hbm,
                            acc_sh.at[pl.ds(s * NSL + k * 128, 128)])
        plsc.subcore_barrier()

        for b in range(NBUF):
            pltpu.async_copy(tab_sh.at[src_v.at[b]], bufs[b], gsem[b])

        def body(gi, carry):
            j0 = gi * NBUF
            for b in range(NBUF):
                j = j0 + b
                pltpu.make_async_copy(tab_sh.at[src_v.at[j]], bufs[b],
                                      gsem[b]).wait()
                pltpu.async_copy(bufs[b], acc_sh.at[dst_v.at[j]], ssem[b],
                                 add=True)
            for b in range(NBUF):
                j = j0 + b
                pltpu.make_async_copy(bufs[b], acc_sh.at[dst_v.at[j]],
                                      ssem[b]).wait()
                pltpu.async_copy(tab_sh.at[src_v.at[j + NBUF]], bufs[b],
                                 gsem[b])
            return carry

        lax.fori_loop(0, NG - 1, body, 0)
        j0 = (NG - 1) * NBUF
        for b in range(NBUF):
            j = j0 + b
            pltpu.make_async_copy(tab_sh.at[src_v.at[j]], bufs[b],
                                  gsem[b]).wait()
            pltpu.async_copy(bufs[b], acc_sh.at[dst_v.at[j]], ssem[b],
                             add=True)
        for b in range(NBUF):
            j = j0 + b
            pltpu.make_async_copy(bufs[b], acc_sh.at[dst_v.at[j]],
                                  ssem[b]).wait()
        plsc.subcore_barrier()
        for k in range(NSL // 128):
            pltpu.sync_copy(acc_sh.at[pl.ds(s * NSL + k * 128, 128)],
                            out_hbm.at[pl.ds(s * NSL + k * 128, 128),
                                       pl.ds(c * GW + p * HGW, HGW)])


@functools.partial(
    pl.kernel,
    out_type=[
        jax.ShapeDtypeStruct((APAD, EMBW), _F32),
        jax.ShapeDtypeStruct((APAD, EMBW), _F32),
    ],
    mesh=_mesh,
    compiler_params=_sc_params,
    scratch_types=[
        pltpu.VMEM((ANB, ABLK), _I32),
        pltpu.VMEM((ANB, ABLK), _I32),
        pltpu.VMEM((ABLK, EMBW), _F32),
        pltpu.VMEM((ABLK, EMBW), _F32),
        pltpu.VMEM((ABLK, EMBW), _F32),
        pltpu.VMEM((ABLK, EMBW), _F32),
        pltpu.SemaphoreType.DMA,
        pltpu.SemaphoreType.DMA,
        pltpu.SemaphoreType.DMA,
        pltpu.SemaphoreType.DMA,
    ],
)
def _sc_gather(emb_hbm, aes_hbm, aet_hbm, ees_out, eet_out,
               sidx_v, tidx_v, sb0, sb1, tb0, tb1, m0, m1, m2, m3):
    c = lax.axis_index("c")
    s = lax.axis_index("s")
    w = s * NC + c
    base = w * ASL
    pltpu.sync_copy(aes_hbm.at[w], sidx_v)
    pltpu.sync_copy(aet_hbm.at[w], tidx_v)
    sb = (sb0, sb1)
    tb = (tb0, tb1)
    sm = (m0, m1)
    tm = (m2, m3)
    pltpu.async_copy(emb_hbm.at[sidx_v.at[0]], sb[0], sm[0])
    pltpu.async_copy(emb_hbm.at[tidx_v.at[0]], tb[0], tm[0])
    for j in range(ANB):
        p = j % 2
        q = (j + 1) % 2
        off = base + j * ABLK
        pltpu.make_async_copy(emb_hbm.at[sidx_v.at[j]], sb[p], sm[p]).wait()
        pltpu.make_async_copy(emb_hbm.at[tidx_v.at[j]], tb[p], tm[p]).wait()
        if j + 1 < ANB:
            pltpu.async_copy(emb_hbm.at[sidx_v.at[j + 1]], sb[q], sm[q])
            pltpu.async_copy(emb_hbm.at[tidx_v.at[j + 1]], tb[q], tm[q])
        pltpu.sync_copy(sb[p], ees_out.at[pl.ds(off, ABLK)])
        pltpu.sync_copy(tb[p], eet_out.at[pl.ds(off, ABLK)])


def _ln_blk(t, g, b, eps=1e-5):
    mu = jnp.mean(t, axis=-1, keepdims=True)
    var = jnp.mean((t - mu) ** 2, axis=-1, keepdims=True)
    return (t - mu) * lax.rsqrt(var + eps) * g + b


_RB = 2048


def _full(shape):
    nd = len(shape)
    return pl.BlockSpec(shape, lambda i, _n=nd: (0,) * _n)


def _rows(bshape):
    return pl.BlockSpec(bshape, lambda i: (i,) + (0,) * (len(bshape) - 1))


def _tc_pre_body(x_ref, w0_ref, pj_ref, u0_ref, xp_ref):
    x = x_ref[...]
    u0_ref[...] = jnp.dot(x, w0_ref[...], preferred_element_type=_F32)
    xp_ref[...] = jnp.dot(x, pj_ref[...], preferred_element_type=_F32)


def _tc_pre(x, w0, proj):
    return pl.pallas_call(
        _tc_pre_body,
        grid=(5,),
        in_specs=[_rows((2000, D_IN)), _full((D_IN, GW)), _full((D_IN, D_H))],
        out_specs=[_rows((2000, GW)), _rows((2000, D_H))],
        out_shape=[jax.ShapeDtypeStruct((NPAD, GW), _F32),
                   jax.ShapeDtypeStruct((NPAD, D_H), _F32)],
    )(x, w0, proj)


def _tc_mid0_body(p0_ref, p1_ref, u0_ref, dinv_ref, g0_ref):
    d = lax.rsqrt(1.0 + p0_ref[...] + p1_ref[...])
    dinv_ref[...] = d
    g0_ref[...] = u0_ref[...] * d


def _tc_mid0(p0, p1, u0):
    return pl.pallas_call(
        _tc_mid0_body,
        grid=(NPAD // _RB,),
        in_specs=[_rows((_RB, 1)), _rows((_RB, 1)), _rows((_RB, GW))],
        out_specs=[_rows((_RB, 1)), _rows((_RB, GW))],
        out_shape=[jax.ShapeDtypeStruct((NPAD, 1), _F32),
                   jax.ShapeDtypeStruct((NPAD, GW), _F32)],
    )(p0, p1, u0)


def _tc_post_body(mode, p_ref, g_ref, dinv_ref, id_ref,
                  b_ref, lg_ref, lb_ref, wn_ref, army_ref,
                  pw0, pb0, pg0, pbe0, pw1, pb1, pg1, pbe1, pw2, pb2,
                  h_ref, gn_ref):
    d = dinv_ref[...]
    p = p_ref[...]
    agg = (p[:, :GW] + p[:, GW:] + g_ref[...]) * d + b_ref[...]
    out = _ln_blk(agg, lg_ref[...], lb_ref[...])
    h = jnp.maximum(out + id_ref[...], 0.0)
    if mode == 'last':
        h_ref[...] = jnp.concatenate(
            [h, army_ref[...], jnp.zeros((h.shape[0], EMBW - D_H - 1), _F32)],
            axis=1)
        t = jnp.maximum(_ln_blk(
            jnp.dot(h, pw0[...], preferred_element_type=_F32) + pb0[...],
            pg0[...], pbe0[...]), 0.0)
        t = jnp.maximum(_ln_blk(
            jnp.dot(t, pw1[...], preferred_element_type=_F32) + pb1[...],
            pg1[...], pbe1[...]), 0.0)
        o = jnp.dot(t, pw2[...], preferred_element_type=_F32) + pb2[...]
        gn_ref[...] = jnp.clip(o, -20.0, 20.0)
    else:
        h_ref[...] = h
        gn_ref[...] = jnp.dot(h, wn_ref[...], preferred_element_type=_F32) * d


def _tc_post(p, g, dinv, ident, b, lg, lb, wnext, army=None, pp=None):
    last = wnext is None
    if last:
        wnext = jnp.zeros((D_H, GW), _F32)
        out_specs = [_rows((_RB, EMBW)), _rows((_RB, 1))]
        out_shape = [jax.ShapeDtypeStruct((NPAD, EMBW), _F32),
                     jax.ShapeDtypeStruct((NPAD, 1), _F32)]
        pargs = (army, pp['W0'], pp['b0'][None, :], pp['g0'][None, :],
                 pp['be0'][None, :], pp['W1'], pp['b1'][None, :],
                 pp['g1'][None, :], pp['be1'][None, :], pp['W2'],
                 pp['b2'][None, :])
    else:
        out_specs = [_rows((_RB, D_H)), _rows((_RB, GW))]
        out_shape = [jax.ShapeDtypeStruct((NPAD, D_H), _F32),
                     jax.ShapeDtypeStruct((NPAD, GW), _F32)]
        pargs = (jnp.zeros((NPAD, 1), _F32), jnp.zeros((D_H, D_H), _F32),
                 jnp.zeros((1, D_H), _F32), jnp.ones((1, D_H), _F32),
                 jnp.zeros((1, D_H), _F32), jnp.zeros((D_H, 32), _F32),
                 jnp.zeros((1, 32), _F32), jnp.ones((1, 32), _F32),
                 jnp.zeros((1, 32), _F32), jnp.zeros((32, 1), _F32),
                 jnp.zeros((1, 1), _F32))
    res = pl.pallas_call(
        functools.partial(_tc_post_body, 'last' if last else 'mid'),
        grid=(NPAD // _RB,),
        in_specs=[
            _rows((_RB, 2 * GW)),
            _rows((_RB, GW)), _rows((_RB, 1)), _rows((_RB, D_H)),
            _full((1, D_H)), _full((1, D_H)), _full((1, D_H)),
            _full((D_H, GW)), _rows((_RB, 1)),
            _full((D_H, D_H)), _full((1, D_H)), _full((1, D_H)),
            _full((1, D_H)),
            _full((D_H, 32)), _full((1, 32)), _full((1, 32)), _full((1, 32)),
            _full((32, 1)), _full((1, 1)),
        ],
        out_specs=out_specs,
        out_shape=out_shape,
    )(p, g, dinv, ident, b, lg, lb, wnext, *pargs)
    return res


def _tc_heads_body(es_ref, et_ref, si_ref, ti_ref,
                   ew0a, ew0b, eb0, eg0, ebe0, ew1, eb1, eg1, ebe1, ew2, eb2,
                   aw0a, aw0b, ab0, ag0, abe0, aw1, ab1, ag1, abe1, aw2, ab2,
                   el_ref, al_ref):
    es_full = es_ref[...]
    et_full = et_ref[...]
    es = jnp.clip(es_full[:, :D_H], -10.0, 10.0)
    et = jnp.clip(et_full[:, :D_H], -10.0, 10.0)
    sa = es_full[:, D_H:D_H + 1]
    ta = et_full[:, D_H:D_H + 1]
    t = jnp.dot(es, ew0a[...], preferred_element_type=_F32) \
        + jnp.dot(et, ew0b[...], preferred_element_type=_F32) + eb0[...]
    t = jnp.maximum(_ln_blk(t, eg0[...], ebe0[...]), 0.0)
    t = jnp.dot(t, ew1[...], preferred_element_type=_F32) + eb1[...]
    t = jnp.maximum(_ln_blk(t, eg1[...], ebe1[...]), 0.0)
    el = jnp.dot(t, ew2[...], preferred_element_type=_F32) + eb2[...]
    a = jnp.dot(es, aw0a[...], preferred_element_type=_F32) \
        + jnp.dot(et, aw0b[...], preferred_element_type=_F32) + ab0[...]
    a = jnp.maximum(_ln_blk(a, ag0[...], abe0[...]), 0.0)
    a = jnp.dot(a, aw1[...], preferred_element_type=_F32) + ab1[...]
    a = jnp.maximum(_ln_blk(a, ag1[...], abe1[...]), 0.0)
    al = jnp.dot(a, aw2[...], preferred_element_type=_F32) + ab2[...]

    si = si_ref[...]
    ti = ti_ref[...]
    valid = (si >= 0) & (ti >= 0)
    bad = valid & ((sa <= 2.0) | (ta >= 3.0 * sa))
    inv_self = (si == ti) & valid
    el = el - bad.astype(_F32) * 1.0 - inv_self.astype(_F32) * 100.0
    el_ref[...] = jnp.clip(el, -20.0, 20.0)

    max_send = sa - 1.0
    col = lax.broadcasted_iota(_I32, al.shape, 1).astype(_F32)
    al = jnp.where(col <= max_send, al, -1000000000.0)
    al_ref[...] = jnp.clip(al, -20.0, 20.0)


def _tc_heads(ees, eet, sidx, tidx, pe, pa):
    aw2p = jnp.zeros((D_H, 64), _F32).at[:, :MAX_ARMY].set(pa['W2'])
    ab2p = jnp.zeros((1, 64), _F32).at[0, :MAX_ARMY].set(pa['b2'])
    return pl.pallas_call(
        _tc_heads_body,
        grid=(APAD // _RB,),
        in_specs=[
            _rows((_RB, EMBW)), _rows((_RB, EMBW)),
            _rows((_RB, 1)), _rows((_RB, 1)),
            _full((D_H, D_H)), _full((D_H, D_H)), _full((1, D_H)),
            _full((1, D_H)), _full((1, D_H)),
            _full((D_H, 32)), _full((1, 32)), _full((1, 32)), _full((1, 32)),
            _full((32, 1)), _full((1, 1)),
            _full((D_H, 128)), _full((D_H, 128)), _full((1, 128)),
            _full((1, 128)), _full((1, 128)),
            _full((128, D_H)), _full((1, D_H)), _full((1, D_H)), _full((1, D_H)),
            _full((D_H, 64)), _full((1, 64)),
        ],
        out_specs=[_rows((_RB, 1)), _rows((_RB, 64))],
        out_shape=[jax.ShapeDtypeStruct((APAD, 1), _F32),
                   jax.ShapeDtypeStruct((APAD, 64), _F32)],
    )(ees, eet, sidx, tidx,
      pe['W0'][:D_H], pe['W0'][D_H:], pe['b0'][None, :], pe['g0'][None, :],
      pe['be0'][None, :], pe['W1'], pe['b1'][None, :], pe['g1'][None, :],
      pe['be1'][None, :], pe['W2'], pe['b2'][None, :],
      pa['W0'][:D_H], pa['W0'][D_H:], pa['b0'][None, :], pa['g0'][None, :],
      pa['be0'][None, :], pa['W1'], pa['b1'][None, :], pa['g1'][None, :],
      pa['be1'][None, :], aw2p, ab2p)


def kernel(x, params, edge_index, action_edges, army_counts):
    ei = edge_index.astype(_I32)
    src_r = ei[0].reshape(NW, NB, BLK)
    dst_r = ei[1].reshape(NW, NB, BLK)
    dst_d = ei[1].reshape(NW, DNB, DBLK)
    ae = action_edges.astype(_I32)
    apad = jnp.zeros((APAD - A,), _I32)
    aes_flat = jnp.concatenate([ae[:, 0], apad])
    aet_flat = jnp.concatenate([ae[:, 1], apad])
    aes = aes_flat.reshape(NW, ANB, ABLK)
    aet = aet_flat.reshape(NW, ANB, ABLK)
    army_f = jnp.zeros((NPAD, 1), _F32).at[:N, 0].set(
        army_counts.astype(_F32))

    zeros1 = jnp.zeros((NSL,), _F32)
    zeros2 = jnp.zeros((128, HGW), _F32)

    degp = _sc_deg(dst_d, zeros1)
    u0, xproj = _tc_pre(x, params['gcn0']['W'], params['gcn0']['proj'])
    dinv, g = _tc_mid0(degp[:NPAD][:, None], degp[NPAD:][:, None], u0)

    emb = None
    place2d = None
    ident = xproj
    for i in range(3):
        p = params['gcn%d' % i]
        pn = params['gcn%d' % (i + 1)] if i < 2 else None
        part = _sc_segsum(g, src_r, dst_r, zeros2)
        if pn is not None:
            h, gnext = _tc_post(part, g, dinv, ident,
                                p['b'][None, :], p['ln_g'][None, :],
                                p['ln_b'][None, :], pn['W'])
            ident = h
            g = gnext
        else:
            emb, place2d = _tc_post(part, g, dinv, ident,
                                    p['b'][None, :], p['ln_g'][None, :],
                                    p['ln_b'][None, :], None,
                                    army=army_f, pp=params['place'])

    ees, eet = _sc_gather(emb, aes, aet)
    el2d, al2d = _tc_heads(ees, eet, aes_flat[:, None], aet_flat[:, None],
                           params['edge_scorer'], params['army_scorer'])

    placement = place2d[:N, 0]
    edge_logits = el2d[:A, 0]
    army_logits = al2d[:A, :MAX_ARMY]
    return placement, edge_logits, army_logits


# speedup vs baseline: 1.2789x; 1.2789x over previous
import functools

import jax
import jax.numpy as jnp
from jax import lax
from jax.experimental import pallas as pl
from jax.experimental.pallas import tpu as pltpu
from jax.experimental.pallas import tpu_sc as plsc

N = 10000
E = 320000
A = 20000
D_IN = 128
D_H = 64
GW = 64
EMBW = 128
MAX_ARMY = 50

NC = 2
NS = 16
NW = NC * NS
NPAD = 10240
NSL = NPAD // NS
EPT = E // NW
BLK = 125
NB = EPT // BLK
DBLK = 80
DNB = EPT // DBLK
NBUF = 4
NG = NB // NBUF
APAD = 20480
ASL = APAD // NW
ABLK = 128
ANB = ASL // ABLK

_F32 = jnp.float32
_I32 = jnp.int32

_mesh = plsc.VectorSubcoreMesh(core_axis_name="c", subcore_axis_name="s")
_sc_params = pltpu.CompilerParams(use_tc_tiling_on_sc=False,
                                  needs_layout_passes=False)


@functools.partial(
    pl.kernel,
    out_type=jax.ShapeDtypeStruct((NC * NPAD,), _F32),
    mesh=_mesh,
    compiler_params=_sc_params,
    scratch_types=[
        pltpu.VMEM((DNB, DBLK), _I32),
        pltpu.VMEM((DBLK,), _F32),
        pltpu.VMEM((NSL,), _F32),
        pltpu.VMEM_SHARED((NPAD,), _F32),
    ],
)
def _sc_deg(dst_hbm, zeros1_hbm, out_hbm, idx_v, ones_v, obuf_v, acc_sh):
    c = lax.axis_index("c")
    s = lax.axis_index("s")
    w = s * NC + c
    pltpu.sync_copy(dst_hbm.at[w], idx_v)
    for k in range(DBLK // 16):
        ones_v[pl.ds(k * 16, 16)] = jnp.ones((16,), _F32)
    pltpu.sync_copy(zeros1_hbm, acc_sh.at[pl.ds(s * NSL, NSL)])
    plsc.subcore_barrier()

    def body(j, carry):
        pltpu.sync_copy(ones_v, acc_sh.at[idx_v.at[j]], add=True)
        return carry

    lax.fori_loop(0, DNB, body, 0)
    plsc.subcore_barrier()
    pltpu.sync_copy(acc_sh.at[pl.ds(s * NSL, NSL)], obuf_v)
    pltpu.sync_copy(obuf_v, out_hbm.at[pl.ds(c * NPAD + s * NSL, NSL)])


@functools.partial(
    pl.kernel,
    out_type=jax.ShapeDtypeStruct((NPAD, 2 * GW), _F32),
    mesh=_mesh,
    compiler_params=_sc_params,
    scratch_types=[
        pltpu.VMEM((NB, BLK), _I32),
        pltpu.VMEM((NB, BLK), _I32),
        pltpu.VMEM((BLK, GW), _F32),
        pltpu.VMEM((BLK, GW), _F32),
        pltpu.VMEM((BLK, GW), _F32),
        pltpu.VMEM((BLK, GW), _F32),
        pltpu.VMEM_SHARED((NPAD, GW), _F32),
        pltpu.SemaphoreType.DMA,
        pltpu.SemaphoreType.DMA,
        pltpu.SemaphoreType.DMA,
        pltpu.SemaphoreType.DMA,
        pltpu.SemaphoreType.DMA,
        pltpu.SemaphoreType.DMA,
        pltpu.SemaphoreType.DMA,
        pltpu.SemaphoreType.DMA,
    ],
)
def _sc_segsum(g_hbm, src_hbm, dst_hbm, zeros2_hbm, out_hbm,
               src_v, dst_v, b0, b1, b2, b3, acc_sh,
               gs0, gs1, gs2, gs3, ss0, ss1, ss2, ss3):
    c = lax.axis_index("c")
    s = lax.axis_index("s")
    w = s * NC + c
    bufs = (b0, b1, b2, b3)
    gsem = (gs0, gs1, gs2, gs3)
    ssem = (ss0, ss1, ss2, ss3)
    pltpu.sync_copy(src_hbm.at[w], src_v)
    pltpu.sync_copy(dst_hbm.at[w], dst_v)
    for k in range(NSL // 128):
        pltpu.sync_copy(zeros2_hbm, acc_sh.at[pl.ds(s * NSL + k * 128, 128)])
    plsc.subcore_barrier()

    for b in range(NBUF):
        pltpu.async_copy(g_hbm.at[src_v.at[b]], bufs[b], gsem[b])

    def body(gi, carry):
        j0 = gi * NBUF
        for b in range(NBUF):
            j = j0 + b
            pltpu.make_async_copy(g_hbm.at[src_v.at[j]], bufs[b],
                                  gsem[b]).wait()
            pltpu.async_copy(bufs[b], acc_sh.at[dst_v.at[j]], ssem[b],
                             add=True)
        for b in range(NBUF):
            j = j0 + b
            pltpu.make_async_copy(bufs[b], acc_sh.at[dst_v.at[j]],
                                  ssem[b]).wait()
            pltpu.async_copy(g_hbm.at[src_v.at[j + NBUF]], bufs[b], gsem[b])
        return carry

    lax.fori_loop(0, NG - 1, body, 0)
    j0 = (NG - 1) * NBUF
    for b in range(NBUF):
        j = j0 + b
        pltpu.make_async_copy(g_hbm.at[src_v.at[j]], bufs[b], gsem[b]).wait()
        pltpu.async_copy(bufs[b], acc_sh.at[dst_v.at[j]], ssem[b], add=True)
    for b in range(NBUF):
        j = j0 + b
        pltpu.make_async_copy(bufs[b], acc_sh.at[dst_v.at[j]], ssem[b]).wait()
    plsc.subcore_barrier()
    for k in range(NSL // 128):
        pltpu.sync_copy(acc_sh.at[pl.ds(s * NSL + k * 128, 128)],
                        out_hbm.at[pl.ds(s * NSL + k * 128, 128),
                                   pl.ds(c * GW, GW)])


@functools.partial(
    pl.kernel,
    out_type=[
        jax.ShapeDtypeStruct((APAD, EMBW), _F32),
        jax.ShapeDtypeStruct((APAD, EMBW), _F32),
    ],
    mesh=_mesh,
    compiler_params=_sc_params,
    scratch_types=[
        pltpu.VMEM((ANB, ABLK), _I32),
        pltpu.VMEM((ANB, ABLK), _I32),
        pltpu.VMEM((ABLK, EMBW), _F32),
        pltpu.VMEM((ABLK, EMBW), _F32),
        pltpu.VMEM((ABLK, EMBW), _F32),
        pltpu.VMEM((ABLK, EMBW), _F32),
        pltpu.SemaphoreType.DMA,
        pltpu.SemaphoreType.DMA,
        pltpu.SemaphoreType.DMA,
        pltpu.SemaphoreType.DMA,
        pltpu.SemaphoreType.DMA,
        pltpu.SemaphoreType.DMA,
        pltpu.SemaphoreType.DMA,
        pltpu.SemaphoreType.DMA,
    ],
)
def _sc_gather(emb_hbm, aes_hbm, aet_hbm, ees_out, eet_out,
               sidx_v, tidx_v, sb0, sb1, tb0, tb1,
               gs0, gs1, gt0, gt1, ws0, ws1, wt0, wt1):
    c = lax.axis_index("c")
    s = lax.axis_index("s")
    w = s * NC + c
    base = w * ASL
    pltpu.sync_copy(aes_hbm.at[w], sidx_v)
    pltpu.sync_copy(aet_hbm.at[w], tidx_v)
    sb = (sb0, sb1)
    tb = (tb0, tb1)
    gss = (gs0, gs1)
    gts = (gt0, gt1)
    wss = (ws0, ws1)
    wts = (wt0, wt1)
    pltpu.async_copy(emb_hbm.at[sidx_v.at[0]], sb[0], gss[0])
    pltpu.async_copy(emb_hbm.at[tidx_v.at[0]], tb[0], gts[0])
    for j in range(ANB):
        p = j % 2
        q = (j + 1) % 2
        off = base + j * ABLK
        pltpu.make_async_copy(emb_hbm.at[sidx_v.at[j]], sb[p], gss[p]).wait()
        pltpu.make_async_copy(emb_hbm.at[tidx_v.at[j]], tb[p], gts[p]).wait()
        pltpu.async_copy(sb[p], ees_out.at[pl.ds(off, ABLK)], wss[p])
        pltpu.async_copy(tb[p], eet_out.at[pl.ds(off, ABLK)], wts[p])
        if j + 1 < ANB:
            if j >= 1:
                poff = base + (j - 1) * ABLK
                pltpu.make_async_copy(sb[q], ees_out.at[pl.ds(poff, ABLK)],
                                      wss[q]).wait()
                pltpu.make_async_copy(tb[q], eet_out.at[pl.ds(poff, ABLK)],
                                      wts[q]).wait()
            pltpu.async_copy(emb_hbm.at[sidx_v.at[j + 1]], sb[q], gss[q])
            pltpu.async_copy(emb_hbm.at[tidx_v.at[j + 1]], tb[q], gts[q])
    for jj in (ANB - 2, ANB - 1):
        pp = jj % 2
        off = base + jj * ABLK
        pltpu.make_async_copy(sb[pp], ees_out.at[pl.ds(off, ABLK)],
                              wss[pp]).wait()
        pltpu.make_async_copy(tb[pp], eet_out.at[pl.ds(off, ABLK)],
                              wts[pp]).wait()


def _ln_blk(t, g, b, eps=1e-5):
    mu = jnp.mean(t, axis=-1, keepdims=True)
    var = jnp.mean((t - mu) ** 2, axis=-1, keepdims=True)
    return (t - mu) * lax.rsqrt(var + eps) * g + b


_RB = 2048


def _full(shape):
    nd = len(shape)
    return pl.BlockSpec(shape, lambda i, _n=nd: (0,) * _n)


def _rows(bshape):
    return pl.BlockSpec(bshape, lambda i: (i,) + (0,) * (len(bshape) - 1))


def _tc_pre_body(x_ref, w0_ref, pj_ref, u0_ref, xp_ref):
    x = x_ref[...]
    u0_ref[...] = jnp.dot(x, w0_ref[...], preferred_element_type=_F32)
    xp_ref[...] = jnp.dot(x, pj_ref[...], preferred_element_type=_F32)


def _tc_pre(x, w0, proj):
    return pl.pallas_call(
        _tc_pre_body,
        grid=(5,),
        in_specs=[_rows((2000, D_IN)), _full((D_IN, GW)), _full((D_IN, D_H))],
        out_specs=[_rows((2000, GW)), _rows((2000, D_H))],
        out_shape=[jax.ShapeDtypeStruct((NPAD, GW), _F32),
                   jax.ShapeDtypeStruct((NPAD, D_H), _F32)],
    )(x, w0, proj)


def _tc_mid0_body(p0_ref, p1_ref, u0_ref, dinv_ref, g0_ref):
    d = lax.rsqrt(1.0 + p0_ref[...] + p1_ref[...])
    dinv_ref[...] = d
    g0_ref[...] = u0_ref[...] * d


def _tc_mid0(p0, p1, u0):
    return pl.pallas_call(
        _tc_mid0_body,
        grid=(NPAD // _RB,),
        in_specs=[_rows((_RB, 1)), _rows((_RB, 1)), _rows((_RB, GW))],
        out_specs=[_rows((_RB, 1)), _rows((_RB, GW))],
        out_shape=[jax.ShapeDtypeStruct((NPAD, 1), _F32),
                   jax.ShapeDtypeStruct((NPAD, GW), _F32)],
    )(p0, p1, u0)


def _tc_post_body(mode, p_ref, g_ref, dinv_ref, id_ref,
                  b_ref, lg_ref, lb_ref, wn_ref, army_ref,
                  pw0, pb0, pg0, pbe0, pw1, pb1, pg1, pbe1, pw2, pb2,
                  h_ref, gn_ref):
    d = dinv_ref[...]
    p = p_ref[...]
    agg = (p[:, :GW] + p[:, GW:] + g_ref[...]) * d + b_ref[...]
    out = _ln_blk(agg, lg_ref[...], lb_ref[...])
    h = jnp.maximum(out + id_ref[...], 0.0)
    if mode == 'last':
        h_ref[...] = jnp.concatenate(
            [h, army_ref[...], jnp.zeros((h.shape[0], EMBW - D_H - 1), _F32)],
            axis=1)
        t = jnp.maximum(_ln_blk(
            jnp.dot(h, pw0[...], preferred_element_type=_F32) + pb0[...],
            pg0[...], pbe0[...]), 0.0)
        t = jnp.maximum(_ln_blk(
            jnp.dot(t, pw1[...], preferred_element_type=_F32) + pb1[...],
            pg1[...], pbe1[...]), 0.0)
        o = jnp.dot(t, pw2[...], preferred_element_type=_F32) + pb2[...]
        gn_ref[...] = jnp.clip(o, -20.0, 20.0)
    else:
        h_ref[...] = h
        gn_ref[...] = jnp.dot(h, wn_ref[...], preferred_element_type=_F32) * d


def _tc_post(p, g, dinv, ident, b, lg, lb, wnext, army=None, pp=None):
    last = wnext is None
    if last:
        wnext = jnp.zeros((D_H, GW), _F32)
        out_specs = [_rows((_RB, EMBW)), _rows((_RB, 1))]
        out_shape = [jax.ShapeDtypeStruct((NPAD, EMBW), _F32),
                     jax.ShapeDtypeStruct((NPAD, 1), _F32)]
        pargs = (army, pp['W0'], pp['b0'][None, :], pp['g0'][None, :],
                 pp['be0'][None, :], pp['W1'], pp['b1'][None, :],
                 pp['g1'][None, :], pp['be1'][None, :], pp['W2'],
                 pp['b2'][None, :])
    else:
        out_specs = [_rows((_RB, D_H)), _rows((_RB, GW))]
        out_shape = [jax.ShapeDtypeStruct((NPAD, D_H), _F32),
                     jax.ShapeDtypeStruct((NPAD, GW), _F32)]
        pargs = (jnp.zeros((NPAD, 1), _F32), jnp.zeros((D_H, D_H), _F32),
                 jnp.zeros((1, D_H), _F32), jnp.ones((1, D_H), _F32),
                 jnp.zeros((1, D_H), _F32), jnp.zeros((D_H, 32), _F32),
                 jnp.zeros((1, 32), _F32), jnp.ones((1, 32), _F32),
                 jnp.zeros((1, 32), _F32), jnp.zeros((32, 1), _F32),
                 jnp.zeros((1, 1), _F32))
    res = pl.pallas_call(
        functools.partial(_tc_post_body, 'last' if last else 'mid'),
        grid=(NPAD // _RB,),
        in_specs=[
            _rows((_RB, 2 * GW)),
            _rows((_RB, GW)), _rows((_RB, 1)), _rows((_RB, D_H)),
            _full((1, D_H)), _full((1, D_H)), _full((1, D_H)),
            _full((D_H, GW)), _rows((_RB, 1)),
            _full((D_H, D_H)), _full((1, D_H)), _full((1, D_H)),
            _full((1, D_H)),
            _full((D_H, 32)), _full((1, 32)), _full((1, 32)), _full((1, 32)),
            _full((32, 1)), _full((1, 1)),
        ],
        out_specs=out_specs,
        out_shape=out_shape,
    )(p, g, dinv, ident, b, lg, lb, wnext, *pargs)
    return res


def _tc_heads_body(es_ref, et_ref, si_ref, ti_ref,
                   ew0a, ew0b, eb0, eg0, ebe0, ew1, eb1, eg1, ebe1, ew2, eb2,
                   aw0a, aw0b, ab0, ag0, abe0, aw1, ab1, ag1, abe1, aw2, ab2,
                   el_ref, al_ref):
    es_full = es_ref[...]
    et_full = et_ref[...]
    es = jnp.clip(es_full[:, :D_H], -10.0, 10.0)
    et = jnp.clip(et_full[:, :D_H], -10.0, 10.0)
    sa = es_full[:, D_H:D_H + 1]
    ta = et_full[:, D_H:D_H + 1]
    t = jnp.dot(es, ew0a[...], preferred_element_type=_F32) \
        + jnp.dot(et, ew0b[...], preferred_element_type=_F32) + eb0[...]
    t = jnp.maximum(_ln_blk(t, eg0[...], ebe0[...]), 0.0)
    t = jnp.dot(t, ew1[...], preferred_element_type=_F32) + eb1[...]
    t = jnp.maximum(_ln_blk(t, eg1[...], ebe1[...]), 0.0)
    el = jnp.dot(t, ew2[...], preferred_element_type=_F32) + eb2[...]
    a = jnp.dot(es, aw0a[...], preferred_element_type=_F32) \
        + jnp.dot(et, aw0b[...], preferred_element_type=_F32) + ab0[...]
    a = jnp.maximum(_ln_blk(a, ag0[...], abe0[...]), 0.0)
    a = jnp.dot(a, aw1[...], preferred_element_type=_F32) + ab1[...]
    a = jnp.maximum(_ln_blk(a, ag1[...], abe1[...]), 0.0)
    al = jnp.dot(a, aw2[...], preferred_element_type=_F32) + ab2[...]

    si = si_ref[...]
    ti = ti_ref[...]
    valid = (si >= 0) & (ti >= 0)
    bad = valid & ((sa <= 2.0) | (ta >= 3.0 * sa))
    inv_self = (si == ti) & valid
    el = el - bad.astype(_F32) * 1.0 - inv_self.astype(_F32) * 100.0
    el_ref[...] = jnp.clip(el, -20.0, 20.0)

    max_send = sa - 1.0
    col = lax.broadcasted_iota(_I32, al.shape, 1).astype(_F32)
    al = jnp.where(col <= max_send, al, -1000000000.0)
    al_ref[...] = jnp.clip(al, -20.0, 20.0)


def _tc_heads(ees, eet, sidx, tidx, pe, pa):
    aw2p = jnp.zeros((D_H, 64), _F32).at[:, :MAX_ARMY].set(pa['W2'])
    ab2p = jnp.zeros((1, 64), _F32).at[0, :MAX_ARMY].set(pa['b2'])
    return pl.pallas_call(
        _tc_heads_body,
        grid=(APAD // _RB,),
        in_specs=[
            _rows((_RB, EMBW)), _rows((_RB, EMBW)),
            _rows((_RB, 1)), _rows((_RB, 1)),
            _full((D_H, D_H)), _full((D_H, D_H)), _full((1, D_H)),
            _full((1, D_H)), _full((1, D_H)),
            _full((D_H, 32)), _full((1, 32)), _full((1, 32)), _full((1, 32)),
            _full((32, 1)), _full((1, 1)),
            _full((D_H, 128)), _full((D_H, 128)), _full((1, 128)),
            _full((1, 128)), _full((1, 128)),
            _full((128, D_H)), _full((1, D_H)), _full((1, D_H)), _full((1, D_H)),
            _full((D_H, 64)), _full((1, 64)),
        ],
        out_specs=[_rows((_RB, 1)), _rows((_RB, 64))],
        out_shape=[jax.ShapeDtypeStruct((APAD, 1), _F32),
                   jax.ShapeDtypeStruct((APAD, 64), _F32)],
    )(ees, eet, sidx, tidx,
      pe['W0'][:D_H], pe['W0'][D_H:], pe['b0'][None, :], pe['g0'][None, :],
      pe['be0'][None, :], pe['W1'], pe['b1'][None, :], pe['g1'][None, :],
      pe['be1'][None, :], pe['W2'], pe['b2'][None, :],
      pa['W0'][:D_H], pa['W0'][D_H:], pa['b0'][None, :], pa['g0'][None, :],
      pa['be0'][None, :], pa['W1'], pa['b1'][None, :], pa['g1'][None, :],
      pa['be1'][None, :], aw2p, ab2p)


def kernel(x, params, edge_index, action_edges, army_counts):
    ei = edge_index.astype(_I32)
    src_r = ei[0].reshape(NW, NB, BLK)
    dst_r = ei[1].reshape(NW, NB, BLK)
    dst_d = ei[1].reshape(NW, DNB, DBLK)
    ae = action_edges.astype(_I32)
    apad = jnp.zeros((APAD - A,), _I32)
    aes_flat = jnp.concatenate([ae[:, 0], apad])
    aet_flat = jnp.concatenate([ae[:, 1], apad])
    aes = aes_flat.reshape(NW, ANB, ABLK)
    aet = aet_flat.reshape(NW, ANB, ABLK)
    army_f = jnp.zeros((NPAD, 1), _F32).at[:N, 0].set(
        army_counts.astype(_F32))

    zeros1 = jnp.zeros((NSL,), _F32)
    zeros2 = jnp.zeros((128, GW), _F32)

    degp = _sc_deg(dst_d, zeros1)
    u0, xproj = _tc_pre(x, params['gcn0']['W'], params['gcn0']['proj'])
    dinv, g = _tc_mid0(degp[:NPAD][:, None], degp[NPAD:][:, None], u0)

    emb = None
    place2d = None
    ident = xproj
    for i in range(3):
        p = params['gcn%d' % i]
        pn = params['gcn%d' % (i + 1)] if i < 2 else None
        part = _sc_segsum(g, src_r, dst_r, zeros2)
        if pn is not None:
            h, gnext = _tc_post(part, g, dinv, ident,
                                p['b'][None, :], p['ln_g'][None, :],
                                p['ln_b'][None, :], pn['W'])
            ident = h
            g = gnext
        else:
            emb, place2d = _tc_post(part, g, dinv, ident,
                                    p['b'][None, :], p['ln_g'][None, :],
                                    p['ln_b'][None, :], None,
                                    army=army_f, pp=params['place'])

    ees, eet = _sc_gather(emb, aes, aet)
    el2d, al2d = _tc_heads(ees, eet, aes_flat[:, None], aet_flat[:, None],
                           params['edge_scorer'], params['army_scorer'])

    placement = place2d[:N, 0]
    edge_logits = el2d[:A, 0]
    army_logits = al2d[:A, :MAX_ARMY]
    return placement, edge_logits, army_logits


# speedup vs baseline: 1.3114x; 1.0254x over previous
import functools

import jax
import jax.numpy as jnp
from jax import lax
from jax.experimental import pallas as pl
from jax.experimental.pallas import tpu as pltpu
from jax.experimental.pallas import tpu_sc as plsc

N = 10000
E = 320000
A = 20000
D_IN = 128
D_H = 64
GW = 64
EMBW = 128
MAX_ARMY = 50

NC = 2
NS = 16
NW = NC * NS
NPAD = 10240
NSL = NPAD // NS
EPT = E // NW
BLK = 125
NB = EPT // BLK
DBLK = 80
DNB = EPT // DBLK
NBUF = 8
NG = NB // NBUF
APAD = 20480
ASL = APAD // NW
ABLK = 128
ANB = ASL // ABLK

_F32 = jnp.float32
_I32 = jnp.int32

_mesh = plsc.VectorSubcoreMesh(core_axis_name="c", subcore_axis_name="s")
_sc_params = pltpu.CompilerParams(use_tc_tiling_on_sc=False,
                                  needs_layout_passes=False)


@functools.partial(
    pl.kernel,
    out_type=jax.ShapeDtypeStruct((NC * NPAD,), _F32),
    mesh=_mesh,
    compiler_params=_sc_params,
    scratch_types=[
        pltpu.VMEM((DNB, DBLK), _I32),
        pltpu.VMEM((DBLK,), _F32),
        pltpu.VMEM((NSL,), _F32),
        pltpu.VMEM_SHARED((NPAD,), _F32),
    ],
)
def _sc_deg(dst_hbm, zeros1_hbm, out_hbm, idx_v, ones_v, obuf_v, acc_sh):
    c = lax.axis_index("c")
    s = lax.axis_index("s")
    w = s * NC + c
    pltpu.sync_copy(dst_hbm.at[w], idx_v)
    for k in range(DBLK // 16):
        ones_v[pl.ds(k * 16, 16)] = jnp.ones((16,), _F32)
    pltpu.sync_copy(zeros1_hbm, acc_sh.at[pl.ds(s * NSL, NSL)])
    plsc.subcore_barrier()

    def body(j, carry):
        pltpu.sync_copy(ones_v, acc_sh.at[idx_v.at[j]], add=True)
        return carry

    lax.fori_loop(0, DNB, body, 0)
    plsc.subcore_barrier()
    pltpu.sync_copy(acc_sh.at[pl.ds(s * NSL, NSL)], obuf_v)
    pltpu.sync_copy(obuf_v, out_hbm.at[pl.ds(c * NPAD + s * NSL, NSL)])


@functools.partial(
    pl.kernel,
    out_type=jax.ShapeDtypeStruct((NPAD, 2 * GW), _F32),
    mesh=_mesh,
    compiler_params=_sc_params,
    scratch_types=[
        pltpu.VMEM((NB, BLK), _I32),
        pltpu.VMEM((NB, BLK), _I32),
        pltpu.VMEM((BLK, GW), _F32),
        pltpu.VMEM((BLK, GW), _F32),
        pltpu.VMEM((BLK, GW), _F32),
        pltpu.VMEM((BLK, GW), _F32),
        pltpu.VMEM((BLK, GW), _F32),
        pltpu.VMEM((BLK, GW), _F32),
        pltpu.VMEM((BLK, GW), _F32),
        pltpu.VMEM((BLK, GW), _F32),
        pltpu.VMEM_SHARED((NPAD, GW), _F32),
        pltpu.SemaphoreType.DMA,
        pltpu.SemaphoreType.DMA,
        pltpu.SemaphoreType.DMA,
        pltpu.SemaphoreType.DMA,
        pltpu.SemaphoreType.DMA,
        pltpu.SemaphoreType.DMA,
        pltpu.SemaphoreType.DMA,
        pltpu.SemaphoreType.DMA,
        pltpu.SemaphoreType.DMA,
        pltpu.SemaphoreType.DMA,
        pltpu.SemaphoreType.DMA,
        pltpu.SemaphoreType.DMA,
        pltpu.SemaphoreType.DMA,
        pltpu.SemaphoreType.DMA,
        pltpu.SemaphoreType.DMA,
        pltpu.SemaphoreType.DMA,
    ],
)
def _sc_segsum(g_hbm, src_hbm, dst_hbm, zeros2_hbm, out_hbm,
               src_v, dst_v, b0, b1, b2, b3, b4, b5, b6, b7, acc_sh,
               gs0, gs1, gs2, gs3, gs4, gs5, gs6, gs7,
               ss0, ss1, ss2, ss3, ss4, ss5, ss6, ss7):
    c = lax.axis_index("c")
    s = lax.axis_index("s")
    w = s * NC + c
    bufs = (b0, b1, b2, b3, b4, b5, b6, b7)
    gsem = (gs0, gs1, gs2, gs3, gs4, gs5, gs6, gs7)
    ssem = (ss0, ss1, ss2, ss3, ss4, ss5, ss6, ss7)
    pltpu.sync_copy(src_hbm.at[w], src_v)
    pltpu.sync_copy(dst_hbm.at[w], dst_v)
    for k in range(NSL // 128):
        pltpu.sync_copy(zeros2_hbm, acc_sh.at[pl.ds(s * NSL + k * 128, 128)])
    plsc.subcore_barrier()

    for b in range(NBUF):
        pltpu.async_copy(g_hbm.at[src_v.at[b]], bufs[b], gsem[b])

    def body(gi, carry):
        j0 = gi * NBUF
        for b in range(NBUF):
            j = j0 + b
            pltpu.make_async_copy(g_hbm.at[src_v.at[j]], bufs[b],
                                  gsem[b]).wait()
            pltpu.async_copy(bufs[b], acc_sh.at[dst_v.at[j]], ssem[b],
                             add=True)
        for b in range(NBUF):
            j = j0 + b
            pltpu.make_async_copy(bufs[b], acc_sh.at[dst_v.at[j]],
                                  ssem[b]).wait()
            pltpu.async_copy(g_hbm.at[src_v.at[j + NBUF]], bufs[b], gsem[b])
        return carry

    lax.fori_loop(0, NG - 1, body, 0)
    j0 = (NG - 1) * NBUF
    for b in range(NBUF):
        j = j0 + b
        pltpu.make_async_copy(g_hbm.at[src_v.at[j]], bufs[b], gsem[b]).wait()
        pltpu.async_copy(bufs[b], acc_sh.at[dst_v.at[j]], ssem[b], add=True)
    for b in range(NBUF):
        j = j0 + b
        pltpu.make_async_copy(bufs[b], acc_sh.at[dst_v.at[j]], ssem[b]).wait()
    plsc.subcore_barrier()
    for k in range(NSL // 128):
        pltpu.sync_copy(acc_sh.at[pl.ds(s * NSL + k * 128, 128)],
                        out_hbm.at[pl.ds(s * NSL + k * 128, 128),
                                   pl.ds(c * GW, GW)])


@functools.partial(
    pl.kernel,
    out_type=[
        jax.ShapeDtypeStruct((APAD, EMBW), _F32),
        jax.ShapeDtypeStruct((APAD, EMBW), _F32),
    ],
    mesh=_mesh,
    compiler_params=_sc_params,
    scratch_types=[
        pltpu.VMEM((ANB, ABLK), _I32),
        pltpu.VMEM((ANB, ABLK), _I32),
        pltpu.VMEM((ABLK, EMBW), _F32),
        pltpu.VMEM((ABLK, EMBW), _F32),
        pltpu.VMEM((ABLK, EMBW), _F32),
        pltpu.VMEM((ABLK, EMBW), _F32),
        pltpu.SemaphoreType.DMA,
        pltpu.SemaphoreType.DMA,
        pltpu.SemaphoreType.DMA,
        pltpu.SemaphoreType.DMA,
        pltpu.SemaphoreType.DMA,
        pltpu.SemaphoreType.DMA,
        pltpu.SemaphoreType.DMA,
        pltpu.SemaphoreType.DMA,
    ],
)
def _sc_gather(emb_hbm, aes_hbm, aet_hbm, ees_out, eet_out,
               sidx_v, tidx_v, sb0, sb1, tb0, tb1,
               gs0, gs1, gt0, gt1, ws0, ws1, wt0, wt1):
    c = lax.axis_index("c")
    s = lax.axis_index("s")
    w = s * NC + c
    base = w * ASL
    pltpu.sync_copy(aes_hbm.at[w], sidx_v)
    pltpu.sync_copy(aet_hbm.at[w], tidx_v)
    sb = (sb0, sb1)
    tb = (tb0, tb1)
    gss = (gs0, gs1)
    gts = (gt0, gt1)
    wss = (ws0, ws1)
    wts = (wt0, wt1)
    pltpu.async_copy(emb_hbm.at[sidx_v.at[0]], sb[0], gss[0])
    pltpu.async_copy(emb_hbm.at[tidx_v.at[0]], tb[0], gts[0])
    for j in range(ANB):
        p = j % 2
        q = (j + 1) % 2
        off = base + j * ABLK
        pltpu.make_async_copy(emb_hbm.at[sidx_v.at[j]], sb[p], gss[p]).wait()
        pltpu.make_async_copy(emb_hbm.at[tidx_v.at[j]], tb[p], gts[p]).wait()
        pltpu.async_copy(sb[p], ees_out.at[pl.ds(off, ABLK)], wss[p])
        pltpu.async_copy(tb[p], eet_out.at[pl.ds(off, ABLK)], wts[p])
        if j + 1 < ANB:
            if j >= 1:
                poff = base + (j - 1) * ABLK
                pltpu.make_async_copy(sb[q], ees_out.at[pl.ds(poff, ABLK)],
                                      wss[q]).wait()
                pltpu.make_async_copy(tb[q], eet_out.at[pl.ds(poff, ABLK)],
                                      wts[q]).wait()
            pltpu.async_copy(emb_hbm.at[sidx_v.at[j + 1]], sb[q], gss[q])
            pltpu.async_copy(emb_hbm.at[tidx_v.at[j + 1]], tb[q], gts[q])
    for jj in (ANB - 2, ANB - 1):
        pp = jj % 2
        off = base + jj * ABLK
        pltpu.make_async_copy(sb[pp], ees_out.at[pl.ds(off, ABLK)],
                              wss[pp]).wait()
        pltpu.make_async_copy(tb[pp], eet_out.at[pl.ds(off, ABLK)],
                              wts[pp]).wait()


def _ln_blk(t, g, b, eps=1e-5):
    mu = jnp.mean(t, axis=-1, keepdims=True)
    var = jnp.mean((t - mu) ** 2, axis=-1, keepdims=True)
    return (t - mu) * lax.rsqrt(var + eps) * g + b


_RB = 2048


def _full(shape):
    nd = len(shape)
    return pl.BlockSpec(shape, lambda i, _n=nd: (0,) * _n)


def _rows(bshape):
    return pl.BlockSpec(bshape, lambda i: (i,) + (0,) * (len(bshape) - 1))


def _tc_pre_body(x_ref, w0_ref, pj_ref, u0_ref, xp_ref):
    x = x_ref[...]
    u0_ref[...] = jnp.dot(x, w0_ref[...], preferred_element_type=_F32)
    xp_ref[...] = jnp.dot(x, pj_ref[...], preferred_element_type=_F32)


def _tc_pre(x, w0, proj):
    return pl.pallas_call(
        _tc_pre_body,
        grid=(5,),
        in_specs=[_rows((2000, D_IN)), _full((D_IN, GW)), _full((D_IN, D_H))],
        out_specs=[_rows((2000, GW)), _rows((2000, D_H))],
        out_shape=[jax.ShapeDtypeStruct((NPAD, GW), _F32),
                   jax.ShapeDtypeStruct((NPAD, D_H), _F32)],
    )(x, w0, proj)


def _tc_mid0_body(p0_ref, p1_ref, u0_ref, dinv_ref, g0_ref):
    d = lax.rsqrt(1.0 + p0_ref[...] + p1_ref[...])
    dinv_ref[...] = d
    g0_ref[...] = u0_ref[...] * d


def _tc_mid0(p0, p1, u0):
    return pl.pallas_call(
        _tc_mid0_body,
        grid=(NPAD // _RB,),
        in_specs=[_rows((_RB, 1)), _rows((_RB, 1)), _rows((_RB, GW))],
        out_specs=[_rows((_RB, 1)), _rows((_RB, GW))],
        out_shape=[jax.ShapeDtypeStruct((NPAD, 1), _F32),
                   jax.ShapeDtypeStruct((NPAD, GW), _F32)],
    )(p0, p1, u0)


def _tc_post_body(mode, p_ref, g_ref, dinv_ref, id_ref,
                  b_ref, lg_ref, lb_ref, wn_ref, army_ref,
                  pw0, pb0, pg0, pbe0, pw1, pb1, pg1, pbe1, pw2, pb2,
                  h_ref, gn_ref):
    d = dinv_ref[...]
    p = p_ref[...]
    agg = (p[:, :GW] + p[:, GW:] + g_ref[...]) * d + b_ref[...]
    out = _ln_blk(agg, lg_ref[...], lb_ref[...])
    h = jnp.maximum(out + id_ref[...], 0.0)
    if mode == 'last':
        h_ref[...] = jnp.concatenate(
            [h, army_ref[...], jnp.zeros((h.shape[0], EMBW - D_H - 1), _F32)],
            axis=1)
        t = jnp.maximum(_ln_blk(
            jnp.dot(h, pw0[...], preferred_element_type=_F32) + pb0[...],
            pg0[...], pbe0[...]), 0.0)
        t = jnp.maximum(_ln_blk(
            jnp.dot(t, pw1[...], preferred_element_type=_F32) + pb1[...],
            pg1[...], pbe1[...]), 0.0)
        o = jnp.dot(t, pw2[...], preferred_element_type=_F32) + pb2[...]
        gn_ref[...] = jnp.clip(o, -20.0, 20.0)
    else:
        h_ref[...] = h
        gn_ref[...] = jnp.dot(h, wn_ref[...], preferred_element_type=_F32) * d


def _tc_post(p, g, dinv, ident, b, lg, lb, wnext, army=None, pp=None):
    last = wnext is None
    if last:
        wnext = jnp.zeros((D_H, GW), _F32)
        out_specs = [_rows((_RB, EMBW)), _rows((_RB, 1))]
        out_shape = [jax.ShapeDtypeStruct((NPAD, EMBW), _F32),
                     jax.ShapeDtypeStruct((NPAD, 1), _F32)]
        pargs = (army, pp['W0'], pp['b0'][None, :], pp['g0'][None, :],
                 pp['be0'][None, :], pp['W1'], pp['b1'][None, :],
                 pp['g1'][None, :], pp['be1'][None, :], pp['W2'],
                 pp['b2'][None, :])
    else:
        out_specs = [_rows((_RB, D_H)), _rows((_RB, GW))]
        out_shape = [jax.ShapeDtypeStruct((NPAD, D_H), _F32),
                     jax.ShapeDtypeStruct((NPAD, GW), _F32)]
        pargs = (jnp.zeros((NPAD, 1), _F32), jnp.zeros((D_H, D_H), _F32),
                 jnp.zeros((1, D_H), _F32), jnp.ones((1, D_H), _F32),
                 jnp.zeros((1, D_H), _F32), jnp.zeros((D_H, 32), _F32),
                 jnp.zeros((1, 32), _F32), jnp.ones((1, 32), _F32),
                 jnp.zeros((1, 32), _F32), jnp.zeros((32, 1), _F32),
                 jnp.zeros((1, 1), _F32))
    res = pl.pallas_call(
        functools.partial(_tc_post_body, 'last' if last else 'mid'),
        grid=(NPAD // _RB,),
        in_specs=[
            _rows((_RB, 2 * GW)),
            _rows((_RB, GW)), _rows((_RB, 1)), _rows((_RB, D_H)),
            _full((1, D_H)), _full((1, D_H)), _full((1, D_H)),
            _full((D_H, GW)), _rows((_RB, 1)),
            _full((D_H, D_H)), _full((1, D_H)), _full((1, D_H)),
            _full((1, D_H)),
            _full((D_H, 32)), _full((1, 32)), _full((1, 32)), _full((1, 32)),
            _full((32, 1)), _full((1, 1)),
        ],
        out_specs=out_specs,
        out_shape=out_shape,
    )(p, g, dinv, ident, b, lg, lb, wnext, *pargs)
    return res


def _tc_heads_body(es_ref, et_ref, si_ref, ti_ref,
                   ew0a, ew0b, eb0, eg0, ebe0, ew1, eb1, eg1, ebe1, ew2, eb2,
                   aw0a, aw0b, ab0, ag0, abe0, aw1, ab1, ag1, abe1, aw2, ab2,
                   el_ref, al_ref):
    es_full = es_ref[...]
    et_full = et_ref[...]
    es = jnp.clip(es_full[:, :D_H], -10.0, 10.0)
    et = jnp.clip(et_full[:, :D_H], -10.0, 10.0)
    sa = es_full[:, D_H:D_H + 1]
    ta = et_full[:, D_H:D_H + 1]
    t = jnp.dot(es, ew0a[...], preferred_element_type=_F32) \
        + jnp.dot(et, ew0b[...], preferred_element_type=_F32) + eb0[...]
    t = jnp.maximum(_ln_blk(t, eg0[...], ebe0[...]), 0.0)
    t = jnp.dot(t, ew1[...], preferred_element_type=_F32) + eb1[...]
    t = jnp.maximum(_ln_blk(t, eg1[...], ebe1[...]), 0.0)
    el = jnp.dot(t, ew2[...], preferred_element_type=_F32) + eb2[...]
    a = jnp.dot(es, aw0a[...], preferred_element_type=_F32) \
        + jnp.dot(et, aw0b[...], preferred_element_type=_F32) + ab0[...]
    a = jnp.maximum(_ln_blk(a, ag0[...], abe0[...]), 0.0)
    a = jnp.dot(a, aw1[...], preferred_element_type=_F32) + ab1[...]
    a = jnp.maximum(_ln_blk(a, ag1[...], abe1[...]), 0.0)
    al = jnp.dot(a, aw2[...], preferred_element_type=_F32) + ab2[...]

    si = si_ref[...]
    ti = ti_ref[...]
    valid = (si >= 0) & (ti >= 0)
    bad = valid & ((sa <= 2.0) | (ta >= 3.0 * sa))
    inv_self = (si == ti) & valid
    el = el - bad.astype(_F32) * 1.0 - inv_self.astype(_F32) * 100.0
    el_ref[...] = jnp.clip(el, -20.0, 20.0)

    max_send = sa - 1.0
    col = lax.broadcasted_iota(_I32, al.shape, 1).astype(_F32)
    al = jnp.where(col <= max_send, al, -1000000000.0)
    al_ref[...] = jnp.clip(al, -20.0, 20.0)


def _tc_heads(ees, eet, sidx, tidx, pe, pa):
    aw2p = jnp.zeros((D_H, 64), _F32).at[:, :MAX_ARMY].set(pa['W2'])
    ab2p = jnp.zeros((1, 64), _F32).at[0, :MAX_ARMY].set(pa['b2'])
    return pl.pallas_call(
        _tc_heads_body,
        grid=(APAD // _RB,),
        in_specs=[
            _rows((_RB, EMBW)), _rows((_RB, EMBW)),
            _rows((_RB, 1)), _rows((_RB, 1)),
            _full((D_H, D_H)), _full((D_H, D_H)), _full((1, D_H)),
            _full((1, D_H)), _full((1, D_H)),
            _full((D_H, 32)), _full((1, 32)), _full((1, 32)), _full((1, 32)),
            _full((32, 1)), _full((1, 1)),
            _full((D_H, 128)), _full((D_H, 128)), _full((1, 128)),
            _full((1, 128)), _full((1, 128)),
            _full((128, D_H)), _full((1, D_H)), _full((1, D_H)), _full((1, D_H)),
            _full((D_H, 64)), _full((1, 64)),
        ],
        out_specs=[_rows((_RB, 1)), _rows((_RB, 64))],
        out_shape=[jax.ShapeDtypeStruct((APAD, 1), _F32),
                   jax.ShapeDtypeStruct((APAD, 64), _F32)],
    )(ees, eet, sidx, tidx,
      pe['W0'][:D_H], pe['W0'][D_H:], pe['b0'][None, :], pe['g0'][None, :],
      pe['be0'][None, :], pe['W1'], pe['b1'][None, :], pe['g1'][None, :],
      pe['be1'][None, :], pe['W2'], pe['b2'][None, :],
      pa['W0'][:D_H], pa['W0'][D_H:], pa['b0'][None, :], pa['g0'][None, :],
      pa['be0'][None, :], pa['W1'], pa['b1'][None, :], pa['g1'][None, :],
      pa['be1'][None, :], aw2p, ab2p)


def kernel(x, params, edge_index, action_edges, army_counts):
    ei = edge_index.astype(_I32)
    src_r = ei[0].reshape(NW, NB, BLK)
    dst_r = ei[1].reshape(NW, NB, BLK)
    dst_d = ei[1].reshape(NW, DNB, DBLK)
    ae = action_edges.astype(_I32)
    apad = jnp.zeros((APAD - A,), _I32)
    aes_flat = jnp.concatenate([ae[:, 0], apad])
    aet_flat = jnp.concatenate([ae[:, 1], apad])
    aes = aes_flat.reshape(NW, ANB, ABLK)
    aet = aet_flat.reshape(NW, ANB, ABLK)
    army_f = jnp.zeros((NPAD, 1), _F32).at[:N, 0].set(
        army_counts.astype(_F32))

    zeros1 = jnp.zeros((NSL,), _F32)
    zeros2 = jnp.zeros((128, GW), _F32)

    degp = _sc_deg(dst_d, zeros1)
    u0, xproj = _tc_pre(x, params['gcn0']['W'], params['gcn0']['proj'])
    dinv, g = _tc_mid0(degp[:NPAD][:, None], degp[NPAD:][:, None], u0)

    emb = None
    place2d = None
    ident = xproj
    for i in range(3):
        p = params['gcn%d' % i]
        pn = params['gcn%d' % (i + 1)] if i < 2 else None
        part = _sc_segsum(g, src_r, dst_r, zeros2)
        if pn is not None:
            h, gnext = _tc_post(part, g, dinv, ident,
                                p['b'][None, :], p['ln_g'][None, :],
                                p['ln_b'][None, :], pn['W'])
            ident = h
            g = gnext
        else:
            emb, place2d = _tc_post(part, g, dinv, ident,
                                    p['b'][None, :], p['ln_g'][None, :],
                                    p['ln_b'][None, :], None,
                                    army=army_f, pp=params['place'])

    ees, eet = _sc_gather(emb, aes, aet)
    el2d, al2d = _tc_heads(ees, eet, aes_flat[:, None], aet_flat[:, None],
                           params['edge_scorer'], params['army_scorer'])

    placement = place2d[:N, 0]
    edge_logits = el2d[:A, 0]
    army_logits = al2d[:A, :MAX_ARMY]
    return placement, edge_logits, army_logits


# speedup vs baseline: 1.3731x; 1.0471x over previous
import functools

import jax
import jax.numpy as jnp
from jax import lax
from jax.experimental import pallas as pl
from jax.experimental.pallas import tpu as pltpu
from jax.experimental.pallas import tpu_sc as plsc

N = 10000
E = 320000
A = 20000
D_IN = 128
D_H = 64
GW = 64
EMBW = 128
MAX_ARMY = 50

NC = 2
NS = 16
NW = NC * NS
NPAD = 10240
NSL = NPAD // NS
EPT = E // NW
BLK = 125
NB = EPT // BLK
DBLK = 80
DNB = EPT // DBLK
NBUF = 8
NG = NB // NBUF
APAD = 20480
ASL = APAD // NW
ABLK = 128
ANB = ASL // ABLK

_F32 = jnp.float32
_I32 = jnp.int32

_mesh = plsc.VectorSubcoreMesh(core_axis_name="c", subcore_axis_name="s")
_sc_params = pltpu.CompilerParams(use_tc_tiling_on_sc=False,
                                  needs_layout_passes=False)


@functools.partial(
    pl.kernel,
    out_type=jax.ShapeDtypeStruct((NC * NPAD,), _F32),
    mesh=_mesh,
    compiler_params=_sc_params,
    scratch_types=[
        pltpu.VMEM((DNB, DBLK), _I32),
        pltpu.VMEM((DBLK,), _F32),
        pltpu.VMEM((NSL,), _F32),
        pltpu.VMEM_SHARED((NPAD,), _F32),
    ],
)
def _sc_deg(dst_hbm, zeros1_hbm, out_hbm, idx_v, ones_v, obuf_v, acc_sh):
    c = lax.axis_index("c")
    s = lax.axis_index("s")
    w = s * NC + c
    pltpu.sync_copy(dst_hbm.at[w], idx_v)
    for k in range(DBLK // 16):
        ones_v[pl.ds(k * 16, 16)] = jnp.ones((16,), _F32)
    pltpu.sync_copy(zeros1_hbm, acc_sh.at[pl.ds(s * NSL, NSL)])
    plsc.subcore_barrier()

    def body(j, carry):
        pltpu.sync_copy(ones_v, acc_sh.at[idx_v.at[j]], add=True)
        return carry

    lax.fori_loop(0, DNB, body, 0)
    plsc.subcore_barrier()
    pltpu.sync_copy(acc_sh.at[pl.ds(s * NSL, NSL)], obuf_v)
    pltpu.sync_copy(obuf_v, out_hbm.at[pl.ds(c * NPAD + s * NSL, NSL)])


@functools.partial(
    pl.kernel,
    out_type=jax.ShapeDtypeStruct((NPAD, 2 * GW), _F32),
    mesh=_mesh,
    compiler_params=_sc_params,
    scratch_types=[
        pltpu.VMEM((NB, BLK), _I32),
        pltpu.VMEM((NB, BLK), _I32),
        pltpu.VMEM((BLK, GW), _F32),
        pltpu.VMEM((BLK, GW), _F32),
        pltpu.VMEM((BLK, GW), _F32),
        pltpu.VMEM((BLK, GW), _F32),
        pltpu.VMEM((BLK, GW), _F32),
        pltpu.VMEM((BLK, GW), _F32),
        pltpu.VMEM((BLK, GW), _F32),
        pltpu.VMEM((BLK, GW), _F32),
        pltpu.VMEM_SHARED((NPAD, GW), _F32),
        pltpu.SemaphoreType.DMA,
        pltpu.SemaphoreType.DMA,
        pltpu.SemaphoreType.DMA,
        pltpu.SemaphoreType.DMA,
        pltpu.SemaphoreType.DMA,
        pltpu.SemaphoreType.DMA,
        pltpu.SemaphoreType.DMA,
        pltpu.SemaphoreType.DMA,
        pltpu.SemaphoreType.DMA,
        pltpu.SemaphoreType.DMA,
        pltpu.SemaphoreType.DMA,
        pltpu.SemaphoreType.DMA,
        pltpu.SemaphoreType.DMA,
        pltpu.SemaphoreType.DMA,
        pltpu.SemaphoreType.DMA,
        pltpu.SemaphoreType.DMA,
    ],
)
def _sc_segsum(g_hbm, src_hbm, dst_hbm, zeros2_hbm, out_hbm,
               src_v, dst_v, b0, b1, b2, b3, b4, b5, b6, b7, acc_sh,
               gs0, gs1, gs2, gs3, gs4, gs5, gs6, gs7,
               ss0, ss1, ss2, ss3, ss4, ss5, ss6, ss7):
    c = lax.axis_index("c")
    s = lax.axis_index("s")
    w = s * NC + c
    bufs = (b0, b1, b2, b3, b4, b5, b6, b7)
    gsem = (gs0, gs1, gs2, gs3, gs4, gs5, gs6, gs7)
    ssem = (ss0, ss1, ss2, ss3, ss4, ss5, ss6, ss7)
    pltpu.sync_copy(src_hbm.at[w], src_v)
    pltpu.sync_copy(dst_hbm.at[w], dst_v)
    for k in range(NSL // 128):
        pltpu.sync_copy(zeros2_hbm, acc_sh.at[pl.ds(s * NSL + k * 128, 128)])
    plsc.subcore_barrier()

    for b in range(NBUF):
        pltpu.async_copy(g_hbm.at[src_v.at[b]], bufs[b], gsem[b])

    def body(gi, carry):
        j0 = gi * NBUF
        for b in range(NBUF):
            j = j0 + b
            pltpu.make_async_copy(g_hbm.at[src_v.at[j]], bufs[b],
                                  gsem[b]).wait()
            pltpu.async_copy(bufs[b], acc_sh.at[dst_v.at[j]], ssem[b],
                             add=True)
        for b in range(NBUF):
            j = j0 + b
            pltpu.make_async_copy(bufs[b], acc_sh.at[dst_v.at[j]],
                                  ssem[b]).wait()
            pltpu.async_copy(g_hbm.at[src_v.at[j + NBUF]], bufs[b], gsem[b])
        return carry

    lax.fori_loop(0, NG - 1, body, 0)
    j0 = (NG - 1) * NBUF
    for b in range(NBUF):
        j = j0 + b
        pltpu.make_async_copy(g_hbm.at[src_v.at[j]], bufs[b], gsem[b]).wait()
        pltpu.async_copy(bufs[b], acc_sh.at[dst_v.at[j]], ssem[b], add=True)
    for b in range(NBUF):
        j = j0 + b
        pltpu.make_async_copy(bufs[b], acc_sh.at[dst_v.at[j]], ssem[b]).wait()
    plsc.subcore_barrier()
    for k in range(NSL // 128):
        pltpu.sync_copy(acc_sh.at[pl.ds(s * NSL + k * 128, 128)],
                        out_hbm.at[pl.ds(s * NSL + k * 128, 128),
                                   pl.ds(c * GW, GW)])


@functools.partial(
    pl.kernel,
    out_type=[
        jax.ShapeDtypeStruct((APAD, EMBW), _F32),
        jax.ShapeDtypeStruct((APAD, EMBW), _F32),
    ],
    mesh=_mesh,
    compiler_params=_sc_params,
    scratch_types=[
        pltpu.VMEM((ANB, ABLK), _I32),
        pltpu.VMEM((ANB, ABLK), _I32),
        pltpu.VMEM((ABLK, EMBW), _F32),
        pltpu.VMEM((ABLK, EMBW), _F32),
        pltpu.VMEM((ABLK, EMBW), _F32),
        pltpu.VMEM((ABLK, EMBW), _F32),
        pltpu.SemaphoreType.DMA,
        pltpu.SemaphoreType.DMA,
        pltpu.SemaphoreType.DMA,
        pltpu.SemaphoreType.DMA,
        pltpu.SemaphoreType.DMA,
        pltpu.SemaphoreType.DMA,
        pltpu.SemaphoreType.DMA,
        pltpu.SemaphoreType.DMA,
    ],
)
def _sc_gather(emb_hbm, aes_hbm, aet_hbm, ees_out, eet_out,
               sidx_v, tidx_v, sb0, sb1, tb0, tb1,
               gs0, gs1, gt0, gt1, ws0, ws1, wt0, wt1):
    c = lax.axis_index("c")
    s = lax.axis_index("s")
    w = s * NC + c
    base = w * ASL
    pltpu.sync_copy(aes_hbm.at[w], sidx_v)
    pltpu.sync_copy(aet_hbm.at[w], tidx_v)
    sb = (sb0, sb1)
    tb = (tb0, tb1)
    gss = (gs0, gs1)
    gts = (gt0, gt1)
    wss = (ws0, ws1)
    wts = (wt0, wt1)
    pltpu.async_copy(emb_hbm.at[sidx_v.at[0]], sb[0], gss[0])
    pltpu.async_copy(emb_hbm.at[tidx_v.at[0]], tb[0], gts[0])
    for j in range(ANB):
        p = j % 2
        q = (j + 1) % 2
        off = base + j * ABLK
        pltpu.make_async_copy(emb_hbm.at[sidx_v.at[j]], sb[p], gss[p]).wait()
        pltpu.make_async_copy(emb_hbm.at[tidx_v.at[j]], tb[p], gts[p]).wait()
        pltpu.async_copy(sb[p], ees_out.at[pl.ds(off, ABLK)], wss[p])
        pltpu.async_copy(tb[p], eet_out.at[pl.ds(off, ABLK)], wts[p])
        if j + 1 < ANB:
            if j >= 1:
                poff = base + (j - 1) * ABLK
                pltpu.make_async_copy(sb[q], ees_out.at[pl.ds(poff, ABLK)],
                                      wss[q]).wait()
                pltpu.make_async_copy(tb[q], eet_out.at[pl.ds(poff, ABLK)],
                                      wts[q]).wait()
            pltpu.async_copy(emb_hbm.at[sidx_v.at[j + 1]], sb[q], gss[q])
            pltpu.async_copy(emb_hbm.at[tidx_v.at[j + 1]], tb[q], gts[q])
    for jj in (ANB - 2, ANB - 1):
        pp = jj % 2
        off = base + jj * ABLK
        pltpu.make_async_copy(sb[pp], ees_out.at[pl.ds(off, ABLK)],
                              wss[pp]).wait()
        pltpu.make_async_copy(tb[pp], eet_out.at[pl.ds(off, ABLK)],
                              wts[pp]).wait()


def _ln_blk(t, g, b, eps=1e-5):
    mu = jnp.mean(t, axis=-1, keepdims=True)
    var = jnp.mean((t - mu) ** 2, axis=-1, keepdims=True)
    return (t - mu) * lax.rsqrt(var + eps) * g + b


_RB = 2000


def _full(shape):
    nd = len(shape)
    return pl.BlockSpec(shape, lambda i, _n=nd: (0,) * _n)


def _rows(bshape):
    return pl.BlockSpec(bshape, lambda i: (i,) + (0,) * (len(bshape) - 1))


def _tc_pre_body(x_ref, w0_ref, pj_ref, p0_ref, p1_ref,
                 g0_ref, xp_ref, dinv_ref):
    x = x_ref[...]
    d = lax.rsqrt(1.0 + p0_ref[...] + p1_ref[...])
    dinv_ref[...] = d
    g0_ref[...] = jnp.dot(x, w0_ref[...], preferred_element_type=_F32) * d
    xp_ref[...] = jnp.dot(x, pj_ref[...], preferred_element_type=_F32)


def _tc_pre(x, w0, proj, p0, p1):
    return pl.pallas_call(
        _tc_pre_body,
        grid=(N // _RB,),
        in_specs=[_rows((_RB, D_IN)), _full((D_IN, GW)), _full((D_IN, D_H)),
                  _rows((_RB, 1)), _rows((_RB, 1))],
        out_specs=[_rows((_RB, GW)), _rows((_RB, D_H)), _rows((_RB, 1))],
        out_shape=[jax.ShapeDtypeStruct((N, GW), _F32),
                   jax.ShapeDtypeStruct((N, D_H), _F32),
                   jax.ShapeDtypeStruct((N, 1), _F32)],
    )(x, w0, proj, p0, p1)


def _tc_post_body(mode, p_ref, g_ref, dinv_ref, id_ref,
                  b_ref, lg_ref, lb_ref, wn_ref, army_ref,
                  h_ref, gn_ref):
    d = dinv_ref[...]
    p = p_ref[...]
    agg = (p[:, :GW] + p[:, GW:] + g_ref[...]) * d + b_ref[...]
    out = _ln_blk(agg, lg_ref[...], lb_ref[...])
    h = jnp.maximum(out + id_ref[...], 0.0)
    if mode == 'last':
        h_ref[...] = jnp.concatenate(
            [h, army_ref[...], jnp.zeros((h.shape[0], EMBW - D_H - 1), _F32)],
            axis=1)
    else:
        h_ref[...] = h
        gn_ref[...] = jnp.dot(h, wn_ref[...], preferred_element_type=_F32) * d


def _tc_post(p, g, dinv, ident, b, lg, lb, wnext, army=None):
    last = wnext is None
    if last:
        wnext = jnp.zeros((D_H, GW), _F32)
        out_specs = [_rows((_RB, EMBW)), _rows((_RB, 1))]
        out_shape = [jax.ShapeDtypeStruct((N, EMBW), _F32),
                     jax.ShapeDtypeStruct((N, 1), _F32)]
    else:
        army = jnp.zeros((N, 1), _F32)
        out_specs = [_rows((_RB, D_H)), _rows((_RB, GW))]
        out_shape = [jax.ShapeDtypeStruct((N, D_H), _F32),
                     jax.ShapeDtypeStruct((N, GW), _F32)]
    res = pl.pallas_call(
        functools.partial(_tc_post_body, 'last' if last else 'mid'),
        grid=(N // _RB,),
        in_specs=[
            _rows((_RB, 2 * GW)),
            _rows((_RB, GW)), _rows((_RB, 1)), _rows((_RB, D_H)),
            _full((1, D_H)), _full((1, D_H)), _full((1, D_H)),
            _full((D_H, GW)), _rows((_RB, 1)),
        ],
        out_specs=out_specs,
        out_shape=out_shape,
    )(p, g, dinv, ident, b, lg, lb, wnext, army)
    return res


def _tc_place_body(e_ref, pw0, pb0, pg0, pbe0, pw1, pb1, pg1, pbe1,
                   pw2, pb2, o_ref):
    h = e_ref[...][:, :D_H]
    t = jnp.maximum(_ln_blk(
        jnp.dot(h, pw0[...], preferred_element_type=_F32) + pb0[...],
        pg0[...], pbe0[...]), 0.0)
    t = jnp.maximum(_ln_blk(
        jnp.dot(t, pw1[...], preferred_element_type=_F32) + pb1[...],
        pg1[...], pbe1[...]), 0.0)
    o = jnp.dot(t, pw2[...], preferred_element_type=_F32) + pb2[...]
    o_ref[...] = jnp.clip(o, -20.0, 20.0)


def _tc_place(emb, pp):
    return pl.pallas_call(
        _tc_place_body,
        grid=(N // _RB,),
        in_specs=[
            _rows((_RB, EMBW)),
            _full((D_H, D_H)), _full((1, D_H)), _full((1, D_H)),
            _full((1, D_H)),
            _full((D_H, 32)), _full((1, 32)), _full((1, 32)), _full((1, 32)),
            _full((32, 1)), _full((1, 1)),
        ],
        out_specs=_rows((_RB, 1)),
        out_shape=jax.ShapeDtypeStruct((N, 1), _F32),
    )(emb, pp['W0'], pp['b0'][None, :], pp['g0'][None, :], pp['be0'][None, :],
      pp['W1'], pp['b1'][None, :], pp['g1'][None, :], pp['be1'][None, :],
      pp['W2'], pp['b2'][None, :])


def _tc_heads_body(es_ref, et_ref, si_ref, ti_ref,
                   ew0a, ew0b, eb0, eg0, ebe0, ew1, eb1, eg1, ebe1, ew2, eb2,
                   aw0a, aw0b, ab0, ag0, abe0, aw1, ab1, ag1, abe1, aw2, ab2,
                   el_ref, al_ref):
    es_full = es_ref[...]
    et_full = et_ref[...]
    es = jnp.clip(es_full[:, :D_H], -10.0, 10.0)
    et = jnp.clip(et_full[:, :D_H], -10.0, 10.0)
    sa = es_full[:, D_H:D_H + 1]
    ta = et_full[:, D_H:D_H + 1]
    t = jnp.dot(es, ew0a[...], preferred_element_type=_F32) \
        + jnp.dot(et, ew0b[...], preferred_element_type=_F32) + eb0[...]
    t = jnp.maximum(_ln_blk(t, eg0[...], ebe0[...]), 0.0)
    t = jnp.dot(t, ew1[...], preferred_element_type=_F32) + eb1[...]
    t = jnp.maximum(_ln_blk(t, eg1[...], ebe1[...]), 0.0)
    el = jnp.dot(t, ew2[...], preferred_element_type=_F32) + eb2[...]
    a = jnp.dot(es, aw0a[...], preferred_element_type=_F32) \
        + jnp.dot(et, aw0b[...], preferred_element_type=_F32) + ab0[...]
    a = jnp.maximum(_ln_blk(a, ag0[...], abe0[...]), 0.0)
    a = jnp.dot(a, aw1[...], preferred_element_type=_F32) + ab1[...]
    a = jnp.maximum(_ln_blk(a, ag1[...], abe1[...]), 0.0)
    al = jnp.dot(a, aw2[...], preferred_element_type=_F32) + ab2[...]

    si = si_ref[...]
    ti = ti_ref[...]
    valid = (si >= 0) & (ti >= 0)
    bad = valid & ((sa <= 2.0) | (ta >= 3.0 * sa))
    inv_self = (si == ti) & valid
    el = el - bad.astype(_F32) * 1.0 - inv_self.astype(_F32) * 100.0
    el_ref[...] = jnp.clip(el, -20.0, 20.0)

    max_send = sa - 1.0
    col = lax.broadcasted_iota(_I32, al.shape, 1).astype(_F32)
    al = jnp.where(col <= max_send, al, -1000000000.0)
    al_ref[...] = jnp.clip(al, -20.0, 20.0)[:, :MAX_ARMY]


def _tc_heads(ees, eet, sidx, tidx, pe, pa):
    aw2p = jnp.zeros((D_H, 64), _F32).at[:, :MAX_ARMY].set(pa['W2'])
    ab2p = jnp.zeros((1, 64), _F32).at[0, :MAX_ARMY].set(pa['b2'])
    return pl.pallas_call(
        _tc_heads_body,
        grid=(A // _RB,),
        in_specs=[
            _rows((_RB, EMBW)), _rows((_RB, EMBW)),
            _rows((_RB, 1)), _rows((_RB, 1)),
            _full((D_H, D_H)), _full((D_H, D_H)), _full((1, D_H)),
            _full((1, D_H)), _full((1, D_H)),
            _full((D_H, 32)), _full((1, 32)), _full((1, 32)), _full((1, 32)),
            _full((32, 1)), _full((1, 1)),
            _full((D_H, 128)), _full((D_H, 128)), _full((1, 128)),
            _full((1, 128)), _full((1, 128)),
            _full((128, D_H)), _full((1, D_H)), _full((1, D_H)), _full((1, D_H)),
            _full((D_H, 64)), _full((1, 64)),
        ],
        out_specs=[_rows((_RB, 1)), _rows((_RB, MAX_ARMY))],
        out_shape=[jax.ShapeDtypeStruct((A, 1), _F32),
                   jax.ShapeDtypeStruct((A, MAX_ARMY), _F32)],
    )(ees, eet, sidx, tidx,
      pe['W0'][:D_H], pe['W0'][D_H:], pe['b0'][None, :], pe['g0'][None, :],
      pe['be0'][None, :], pe['W1'], pe['b1'][None, :], pe['g1'][None, :],
      pe['be1'][None, :], pe['W2'], pe['b2'][None, :],
      pa['W0'][:D_H], pa['W0'][D_H:], pa['b0'][None, :], pa['g0'][None, :],
      pa['be0'][None, :], pa['W1'], pa['b1'][None, :], pa['g1'][None, :],
      pa['be1'][None, :], aw2p, ab2p)


def kernel(x, params, edge_index, action_edges, army_counts):
    ei = edge_index.astype(_I32)
    src_r = ei[0].reshape(NW, NB, BLK)
    dst_r = ei[1].reshape(NW, NB, BLK)
    dst_d = ei[1].reshape(NW, DNB, DBLK)
    ae = action_edges.astype(_I32)
    apad = jnp.zeros((APAD - A,), _I32)
    aes_flat = jnp.concatenate([ae[:, 0], apad])
    aet_flat = jnp.concatenate([ae[:, 1], apad])
    aes = aes_flat.reshape(NW, ANB, ABLK)
    aet = aet_flat.reshape(NW, ANB, ABLK)
    army_f = army_counts.astype(_F32)[:, None]

    zeros1 = jnp.zeros((NSL,), _F32)
    zeros2 = jnp.zeros((128, GW), _F32)

    degp = _sc_deg(dst_d, zeros1)
    g, xproj, dinv = _tc_pre(x, params['gcn0']['W'], params['gcn0']['proj'],
                             degp[:NPAD][:, None], degp[NPAD:][:, None])

    emb = None
    ident = xproj
    for i in range(3):
        p = params['gcn%d' % i]
        pn = params['gcn%d' % (i + 1)] if i < 2 else None
        part = _sc_segsum(g, src_r, dst_r, zeros2)
        if pn is not None:
            h, gnext = _tc_post(part, g, dinv, ident,
                                p['b'][None, :], p['ln_g'][None, :],
                                p['ln_b'][None, :], pn['W'])
            ident = h
            g = gnext
        else:
            emb, _ = _tc_post(part, g, dinv, ident,
                              p['b'][None, :], p['ln_g'][None, :],
                              p['ln_b'][None, :], None, army=army_f)

    ees, eet = _sc_gather(emb, aes, aet)
    place2d = _tc_place(emb, params['place'])
    el2d, al2d = _tc_heads(ees, eet, ae[:, 0:1], ae[:, 1:2],
                           params['edge_scorer'], params['army_scorer'])

    placement = place2d[:, 0]
    edge_logits = el2d[:, 0]
    army_logits = al2d
    return placement, edge_logits, army_logits
